# baseline (speedup 1.0000x reference)
import functools
import math

import jax
import jax.numpy as jnp
from jax.experimental import pallas as pl
from jax.experimental.pallas import tpu as pltpu

N, D, P, K = 4096, 512, 16, 128
PD = P * D

BM = 256
BK = 512
KBLKS = PD // BK // 2
BN_X = 256
MBLKS = N // BM


def _build_x_kernel(c_ref, w_ref, hi_ref):
    c = c_ref[...]
    w = w_ref[0]
    cf = c * w
    norm = jnp.sqrt(jnp.sum(cf * cf, axis=1, keepdims=True))
    x = cf / jnp.maximum(norm, 1e-12)
    hi_ref[...] = x.astype(jnp.bfloat16)[None]


def _attn_topk_kernel(lhs_ref, rhs_ref, out_ref, acc_ref, skey_ref, t_ref):
    m = pl.program_id(0)
    k = pl.program_id(1)
    dims = (((1,), (1,)), ((), ()))

    part = jax.lax.dot_general(lhs_ref[0], rhs_ref[0], dims,
                               preferred_element_type=jnp.float32)
    part = part + jax.lax.dot_general(lhs_ref[1], rhs_ref[1], dims,
                                      preferred_element_type=jnp.float32)
    acc_ref[...] = jnp.where(k == 0, part, acc_ref[...] + part)

    t = jnp.where(k == 0,
                  jnp.full((BM, 1), jnp.int32(-2147483648), jnp.int32),
                  t_ref[...])
    for j in range(4):
        i = 4 * k + j
        cand = t + (jnp.int32(1) << (jnp.int32(31) - i))
        cnt = jnp.zeros((BM, 1), jnp.int32)
        for c in range(16):
            chunk = skey_ref[:, c * (N // 16):(c + 1) * (N // 16)]
            cnt = cnt + jnp.sum((chunk >= cand).astype(jnp.int32),
                                axis=1, keepdims=True)
        t = jnp.where(cnt >= K, cand, t)
    t_ref[...] = t

    @pl.when((m > 0) & (k == KBLKS - 1))
    def _():
        sk = skey_ref[...]
        att = jax.lax.bitcast_convert_type(
            sk ^ ((sk >> 31) & jnp.int32(0x7FFFFFFF)), jnp.float32)
        out_ref[...] = jnp.where(sk >= t_ref[...], att, jnp.float32(0.0))

    @pl.when(k == KBLKS - 1)
    def _():
        att = acc_ref[...] * jnp.float32(1.0 / P)
        bits = jax.lax.bitcast_convert_type(att, jnp.int32)
        skey_ref[...] = bits ^ ((bits >> 31) & jnp.int32(0x7FFFFFFF))


@jax.jit
def kernel(context, weight):
    x = pl.pallas_call(
        _build_x_kernel,
        grid=(N // BN_X, P),
        in_specs=[
            pl.BlockSpec((BN_X, D), lambda i, p: (i, 0)),
            pl.BlockSpec((1, 1, D), lambda i, p: (p, 0, 0)),
        ],
        out_specs=pl.BlockSpec((1, BN_X, D), lambda i, p: (p, i, 0)),
        out_shape=jax.ShapeDtypeStruct((P, N, D), jnp.bfloat16),
    )(context, weight.reshape(P, 1, D))

    out = pl.pallas_call(
        _attn_topk_kernel,
        grid=(MBLKS + 1, KBLKS),
        in_specs=[
            pl.BlockSpec((2, BM, BK),
                         lambda m, k: (jnp.where(m < MBLKS, k, 0),
                                       jnp.where(m < MBLKS, m, 0), 0)),
            pl.BlockSpec((2, N, BK),
                         lambda m, k: (jnp.where(m < MBLKS, k, 0), 0, 0)),
        ],
        out_specs=pl.BlockSpec((BM, N),
                               lambda m, k: (jnp.maximum(m - 1, 0), 0)),
        out_shape=jax.ShapeDtypeStruct((N, N), jnp.float32),
        scratch_shapes=[
            pltpu.VMEM((BM, N), jnp.float32),
            pltpu.VMEM((BM, N), jnp.int32),
            pltpu.VMEM((BM, 1), jnp.int32),
        ],
        compiler_params=pltpu.CompilerParams(
            dimension_semantics=("arbitrary", "arbitrary")),
    )(x, x)
    return out


# speedup vs baseline: 1.3689x; 1.3689x over previous
import functools
import math

import jax
import jax.numpy as jnp
from jax.experimental import pallas as pl
from jax.experimental.pallas import tpu as pltpu

N, D, P, K = 4096, 512, 16, 128
PD = P * D

BM = 256
BK = 512
KBLKS = PD // BK // 2
BN_X = 512
MBLKS = N // BM


def _build_x_kernel(c_ref, w_ref, hi_ref):
    c = c_ref[...]
    w = w_ref[0]
    cf = c * w
    norm = jnp.sqrt(jnp.sum(cf * cf, axis=1, keepdims=True))
    x = cf / jnp.maximum(norm, 1e-12)
    hi_ref[...] = x.astype(jnp.bfloat16)[None]


def _attn_topk_kernel(lhs_ref, rhs_ref, out_ref, acc_ref, skey_ref, t_ref):
    m = pl.program_id(0)
    k = pl.program_id(1)
    dims = (((1,), (1,)), ((), ()))

    part = jax.lax.dot_general(lhs_ref[0], rhs_ref[0], dims,
                               preferred_element_type=jnp.float32)
    part = part + jax.lax.dot_general(lhs_ref[1], rhs_ref[1], dims,
                                      preferred_element_type=jnp.float32)
    acc_ref[...] = jnp.where(k == 0, part, acc_ref[...] + part)

    t = jnp.where(k == 0,
                  jnp.full((BM, 1), jnp.int32(-2147483648), jnp.int32),
                  t_ref[...])
    for j in range(4):
        i = 4 * k + j
        cand = t + (jnp.int32(1) << (jnp.int32(31) - i))
        cnt = jnp.zeros((BM, 1), jnp.int32)
        for c in range(8):
            chunk = skey_ref[:, c * (N // 8):(c + 1) * (N // 8)]
            cnt = cnt + jnp.sum((chunk >= cand).astype(jnp.int32),
                                axis=1, keepdims=True)
        t = jnp.where(cnt >= K, cand, t)
    t_ref[...] = t

    @pl.when((m > 0) & (k == KBLKS - 1))
    def _():
        sk = skey_ref[...]
        att = jax.lax.bitcast_convert_type(
            sk ^ ((sk >> 31) & jnp.int32(0x7FFFFFFF)), jnp.float32)
        out_ref[...] = jnp.where(sk >= t_ref[...], att, jnp.float32(0.0))

    @pl.when(k == KBLKS - 1)
    def _():
        att = acc_ref[...] * jnp.float32(1.0 / P)
        bits = jax.lax.bitcast_convert_type(att, jnp.int32)
        skey_ref[...] = bits ^ ((bits >> 31) & jnp.int32(0x7FFFFFFF))


@jax.jit
def kernel(context, weight):
    x = pl.pallas_call(
        _build_x_kernel,
        grid=(N // BN_X, P),
        in_specs=[
            pl.BlockSpec((BN_X, D), lambda i, p: (i, 0)),
            pl.BlockSpec((1, 1, D), lambda i, p: (p, 0, 0)),
        ],
        out_specs=pl.BlockSpec((1, BN_X, D), lambda i, p: (p, i, 0)),
        out_shape=jax.ShapeDtypeStruct((P, N, D), jnp.bfloat16),
    )(context, weight.reshape(P, 1, D))

    out = pl.pallas_call(
        _attn_topk_kernel,
        grid=(MBLKS + 1, KBLKS),
        in_specs=[
            pl.BlockSpec((2, BM, BK),
                         lambda m, k: (jnp.where(m < MBLKS, k, 0),
                                       jnp.where(m < MBLKS, m, 0), 0)),
            pl.BlockSpec((2, N, BK),
                         lambda m, k: (jnp.where(m < MBLKS, k, 0), 0, 0)),
        ],
        out_specs=pl.BlockSpec((BM, N),
                               lambda m, k: (jnp.maximum(m - 1, 0), 0)),
        out_shape=jax.ShapeDtypeStruct((N, N), jnp.float32),
        scratch_shapes=[
            pltpu.VMEM((BM, N), jnp.float32),
            pltpu.VMEM((BM, N), jnp.int32),
            pltpu.VMEM((BM, 1), jnp.int32),
        ],
        compiler_params=pltpu.CompilerParams(
            dimension_semantics=("arbitrary", "arbitrary")),
    )(x, x)
    return out


# speedup vs baseline: 1.4246x; 1.0407x over previous
import functools
import math

import jax
import jax.numpy as jnp
from jax.experimental import pallas as pl
from jax.experimental.pallas import tpu as pltpu

N, D, P, K = 4096, 512, 16, 128
PD = P * D

BM = 256
BK = 512
KBLKS = PD // BK // 2
BN_X = 1024
MBLKS = N // BM


def _build_x_kernel(c_ref, w_ref, hi_ref):
    c = c_ref[...]
    w = w_ref[0]
    cf = c * w
    norm = jnp.sqrt(jnp.sum(cf * cf, axis=1, keepdims=True))
    x = cf / jnp.maximum(norm, 1e-12)
    hi_ref[...] = x.astype(jnp.bfloat16)[None]


def _attn_topk_kernel(lhs_ref, rhs_ref, out_ref, acc_ref, skey_ref, t_ref):
    m = pl.program_id(0)
    k = pl.program_id(1)
    dims = (((1,), (1,)), ((), ()))

    part = jax.lax.dot_general(lhs_ref[0], rhs_ref[0], dims,
                               preferred_element_type=jnp.float32)
    part = part + jax.lax.dot_general(lhs_ref[1], rhs_ref[1], dims,
                                      preferred_element_type=jnp.float32)
    acc_ref[...] = jnp.where(k == 0, part, acc_ref[...] + part)

    t = jnp.where(k == 0,
                  jnp.full((BM, 1), jnp.int32(-2147483648), jnp.int32),
                  t_ref[...])
    for j in range(4):
        i = 4 * k + j
        cand = t + (jnp.int32(1) << (jnp.int32(31) - i))
        cnt = jnp.zeros((BM, 1), jnp.int32)
        for c in range(8):
            chunk = skey_ref[:, c * (N // 8):(c + 1) * (N // 8)]
            cnt = cnt + jnp.sum((chunk >= cand).astype(jnp.int32),
                                axis=1, keepdims=True)
        t = jnp.where(cnt >= K, cand, t)
    t_ref[...] = t

    @pl.when((m > 0) & (k == KBLKS - 1))
    def _():
        sk = skey_ref[...]
        att = jax.lax.bitcast_convert_type(
            sk ^ ((sk >> 31) & jnp.int32(0x7FFFFFFF)), jnp.float32)
        out_ref[...] = jnp.where(sk >= t_ref[...], att, jnp.float32(0.0))

    @pl.when(k == KBLKS - 1)
    def _():
        att = acc_ref[...] * jnp.float32(1.0 / P)
        bits = jax.lax.bitcast_convert_type(att, jnp.int32)
        skey_ref[...] = bits ^ ((bits >> 31) & jnp.int32(0x7FFFFFFF))


@jax.jit
def kernel(context, weight):
    x = pl.pallas_call(
        _build_x_kernel,
        grid=(N // BN_X, P),
        in_specs=[
            pl.BlockSpec((BN_X, D), lambda i, p: (i, 0)),
            pl.BlockSpec((1, 1, D), lambda i, p: (p, 0, 0)),
        ],
        out_specs=pl.BlockSpec((1, BN_X, D), lambda i, p: (p, i, 0)),
        out_shape=jax.ShapeDtypeStruct((P, N, D), jnp.bfloat16),
    )(context, weight.reshape(P, 1, D))

    out = pl.pallas_call(
        _attn_topk_kernel,
        grid=(MBLKS + 1, KBLKS),
        in_specs=[
            pl.BlockSpec((2, BM, BK),
                         lambda m, k: (jnp.where(m < MBLKS, k, 0),
                                       jnp.where(m < MBLKS, m, 0), 0)),
            pl.BlockSpec((2, N, BK),
                         lambda m, k: (jnp.where(m < MBLKS, k, 0), 0, 0)),
        ],
        out_specs=pl.BlockSpec((BM, N),
                               lambda m, k: (jnp.maximum(m - 1, 0), 0)),
        out_shape=jax.ShapeDtypeStruct((N, N), jnp.float32),
        scratch_shapes=[
            pltpu.VMEM((BM, N), jnp.float32),
            pltpu.VMEM((BM, N), jnp.int32),
            pltpu.VMEM((BM, 1), jnp.int32),
        ],
        compiler_params=pltpu.CompilerParams(
            dimension_semantics=("arbitrary", "arbitrary")),
    )(x, x)
    return out


# speedup vs baseline: 1.4556x; 1.0217x over previous
import functools
import math

import jax
import jax.numpy as jnp
from jax.experimental import pallas as pl
from jax.experimental.pallas import tpu as pltpu

N, D, P, K = 4096, 512, 16, 128
PD = P * D

BM = 256
BK = 512
KBLKS = PD // BK // 2
BN_X = 2048
MBLKS = N // BM


def _build_x_kernel(c_ref, w_ref, hi_ref):
    c = c_ref[...]
    w = w_ref[0]
    cf = c * w
    norm = jnp.sqrt(jnp.sum(cf * cf, axis=1, keepdims=True))
    x = cf / jnp.maximum(norm, 1e-12)
    hi_ref[...] = x.astype(jnp.bfloat16)[None]


def _attn_topk_kernel(lhs_ref, rhs_ref, out_ref, acc_ref, skey_ref, t_ref):
    m = pl.program_id(0)
    k = pl.program_id(1)
    dims = (((1,), (1,)), ((), ()))

    part = jax.lax.dot_general(lhs_ref[0], rhs_ref[0], dims,
                               preferred_element_type=jnp.float32)
    part = part + jax.lax.dot_general(lhs_ref[1], rhs_ref[1], dims,
                                      preferred_element_type=jnp.float32)
    acc_ref[...] = jnp.where(k == 0, part, acc_ref[...] + part)

    t = jnp.where(k == 0,
                  jnp.full((BM, 1), jnp.int32(-2147483648), jnp.int32),
                  t_ref[...])
    for j in range(4):
        i = 4 * k + j
        cand = t + (jnp.int32(1) << (jnp.int32(31) - i))
        cnt = jnp.zeros((BM, 1), jnp.int32)
        for c in range(8):
            chunk = skey_ref[:, c * (N // 8):(c + 1) * (N // 8)]
            cnt = cnt + jnp.sum((chunk >= cand).astype(jnp.int32),
                                axis=1, keepdims=True)
        t = jnp.where(cnt >= K, cand, t)
    t_ref[...] = t

    @pl.when((m > 0) & (k == KBLKS - 1))
    def _():
        sk = skey_ref[...]
        att = jax.lax.bitcast_convert_type(
            sk ^ ((sk >> 31) & jnp.int32(0x7FFFFFFF)), jnp.float32)
        out_ref[...] = jnp.where(sk >= t_ref[...], att, jnp.float32(0.0))

    @pl.when(k == KBLKS - 1)
    def _():
        att = acc_ref[...] * jnp.float32(1.0 / P)
        bits = jax.lax.bitcast_convert_type(att, jnp.int32)
        skey_ref[...] = bits ^ ((bits >> 31) & jnp.int32(0x7FFFFFFF))


@jax.jit
def kernel(context, weight):
    x = pl.pallas_call(
        _build_x_kernel,
        grid=(N // BN_X, P),
        in_specs=[
            pl.BlockSpec((BN_X, D), lambda i, p: (i, 0)),
            pl.BlockSpec((1, 1, D), lambda i, p: (p, 0, 0)),
        ],
        out_specs=pl.BlockSpec((1, BN_X, D), lambda i, p: (p, i, 0)),
        out_shape=jax.ShapeDtypeStruct((P, N, D), jnp.bfloat16),
    )(context, weight.reshape(P, 1, D))

    out = pl.pallas_call(
        _attn_topk_kernel,
        grid=(MBLKS + 1, KBLKS),
        in_specs=[
            pl.BlockSpec((2, BM, BK),
                         lambda m, k: (jnp.where(m < MBLKS, k, 0),
                                       jnp.where(m < MBLKS, m, 0), 0)),
            pl.BlockSpec((2, N, BK),
                         lambda m, k: (jnp.where(m < MBLKS, k, 0), 0, 0)),
        ],
        out_specs=pl.BlockSpec((BM, N),
                               lambda m, k: (jnp.maximum(m - 1, 0), 0)),
        out_shape=jax.ShapeDtypeStruct((N, N), jnp.float32),
        scratch_shapes=[
            pltpu.VMEM((BM, N), jnp.float32),
            pltpu.VMEM((BM, N), jnp.int32),
            pltpu.VMEM((BM, 1), jnp.int32),
        ],
        compiler_params=pltpu.CompilerParams(
            dimension_semantics=("arbitrary", "arbitrary")),
    )(x, x)
    return out


# speedup vs baseline: 1.4685x; 1.0089x over previous
import functools
import math

import jax
import jax.numpy as jnp
from jax.experimental import pallas as pl
from jax.experimental.pallas import tpu as pltpu

N, D, P, K = 4096, 512, 16, 128
PD = P * D

BM = 256
BK = 512
KBLKS = PD // BK // 2
BN_X = 4096
MBLKS = N // BM


def _build_x_kernel(c_ref, w_ref, hi_ref):
    c = c_ref[...]
    w = w_ref[0]
    cf = c * w
    norm = jnp.sqrt(jnp.sum(cf * cf, axis=1, keepdims=True))
    x = cf / jnp.maximum(norm, 1e-12)
    hi_ref[...] = x.astype(jnp.bfloat16)[None]


def _attn_topk_kernel(lhs_ref, rhs_ref, out_ref, acc_ref, skey_ref, t_ref):
    m = pl.program_id(0)
    k = pl.program_id(1)
    dims = (((1,), (1,)), ((), ()))

    part = jax.lax.dot_general(lhs_ref[0], rhs_ref[0], dims,
                               preferred_element_type=jnp.float32)
    part = part + jax.lax.dot_general(lhs_ref[1], rhs_ref[1], dims,
                                      preferred_element_type=jnp.float32)
    acc_ref[...] = jnp.where(k == 0, part, acc_ref[...] + part)

    t = jnp.where(k == 0,
                  jnp.full((BM, 1), jnp.int32(-2147483648), jnp.int32),
                  t_ref[...])
    for j in range(4):
        i = 4 * k + j
        cand = t + (jnp.int32(1) << (jnp.int32(31) - i))
        cnt = jnp.zeros((BM, 1), jnp.int32)
        for c in range(8):
            chunk = skey_ref[:, c * (N // 8):(c + 1) * (N // 8)]
            cnt = cnt + jnp.sum((chunk >= cand).astype(jnp.int32),
                                axis=1, keepdims=True)
        t = jnp.where(cnt >= K, cand, t)
    t_ref[...] = t

    @pl.when((m > 0) & (k == KBLKS - 1))
    def _():
        sk = skey_ref[...]
        att = jax.lax.bitcast_convert_type(
            sk ^ ((sk >> 31) & jnp.int32(0x7FFFFFFF)), jnp.float32)
        out_ref[...] = jnp.where(sk >= t_ref[...], att, jnp.float32(0.0))

    @pl.when(k == KBLKS - 1)
    def _():
        att = acc_ref[...] * jnp.float32(1.0 / P)
        bits = jax.lax.bitcast_convert_type(att, jnp.int32)
        skey_ref[...] = bits ^ ((bits >> 31) & jnp.int32(0x7FFFFFFF))


@jax.jit
def kernel(context, weight):
    x = pl.pallas_call(
        _build_x_kernel,
        grid=(N // BN_X, P),
        in_specs=[
            pl.BlockSpec((BN_X, D), lambda i, p: (i, 0)),
            pl.BlockSpec((1, 1, D), lambda i, p: (p, 0, 0)),
        ],
        out_specs=pl.BlockSpec((1, BN_X, D), lambda i, p: (p, i, 0)),
        out_shape=jax.ShapeDtypeStruct((P, N, D), jnp.bfloat16),
    )(context, weight.reshape(P, 1, D))

    out = pl.pallas_call(
        _attn_topk_kernel,
        grid=(MBLKS + 1, KBLKS),
        in_specs=[
            pl.BlockSpec((2, BM, BK),
                         lambda m, k: (jnp.where(m < MBLKS, k, 0),
                                       jnp.where(m < MBLKS, m, 0), 0)),
            pl.BlockSpec((2, N, BK),
                         lambda m, k: (jnp.where(m < MBLKS, k, 0), 0, 0)),
        ],
        out_specs=pl.BlockSpec((BM, N),
                               lambda m, k: (jnp.maximum(m - 1, 0), 0)),
        out_shape=jax.ShapeDtypeStruct((N, N), jnp.float32),
        scratch_shapes=[
            pltpu.VMEM((BM, N), jnp.float32),
            pltpu.VMEM((BM, N), jnp.int32),
            pltpu.VMEM((BM, 1), jnp.int32),
        ],
        compiler_params=pltpu.CompilerParams(
            dimension_semantics=("arbitrary", "arbitrary")),
    )(x, x)
    return out
